# Initial kernel scaffold; baseline (speedup 1.0000x reference)
import jax
import jax.numpy as jnp
from jax.experimental import pallas as pl


def kernel(agent_features, task_features, edge_index, agent_in_w, agent_in_b, task_in_w, task_in_b, gin_w1, gin_b1, gin_w2, gin_b2, agent_out_w, agent_out_b, task_out_w, task_out_b):
    raise NotImplementedError("write your pallas kernel here")


# speedup vs baseline: 2.9978x; 2.9978x over previous
import functools

import jax
import jax.numpy as jnp
from jax import lax
from jax.experimental import pallas as pl
from jax.experimental.pallas import tpu as pltpu
from jax.experimental.pallas import tpu_sc as plsc

NUM_AGENTS = 1000
NUM_TASKS = 9000
N_NODES = NUM_AGENTS + NUM_TASKS
N_EDGES = 320000
AF = 8
TF = 6
H = 128
NUM_LAYERS = 3

NC = 2
NS = 16
NW = NC * NS
CHUNK = 128
EDGES_PER_TILE = 10240
NCHUNK = EDGES_PER_TILE // CHUNK
E_PAD = NW * EDGES_PER_TILE
AGG_ROWS = 10240
ROWS_PER_TILE = AGG_ROWS // NS
TRASH_ROW = N_NODES


def _sc_scatter(x, src_r, dst_r, zeros):
    mesh = plsc.VectorSubcoreMesh(
        core_axis_name="c", subcore_axis_name="s", num_cores=NC, num_subcores=NS
    )

    @functools.partial(
        pl.kernel,
        out_type=jax.ShapeDtypeStruct((NC, AGG_ROWS, H), jnp.float32),
        mesh=mesh,
        scratch_types=[
            pltpu.VMEM((NCHUNK, CHUNK), jnp.int32),
            pltpu.VMEM((NCHUNK, CHUNK), jnp.int32),
            pltpu.VMEM((CHUNK, H), jnp.float32),
            pltpu.VMEM_SHARED((AGG_ROWS, H), jnp.float32),
            pltpu.SemaphoreType.DMA,
        ],
    )
    def k(x_hbm, src_hbm, dst_hbm, zeros_hbm, agg_hbm, src_v, dst_v, rows_v,
          agg_sh, sem):
        c = lax.axis_index("c")
        s = lax.axis_index("s")
        w = c * NS + s
        pltpu.sync_copy(src_hbm.at[w], src_v)
        pltpu.sync_copy(dst_hbm.at[w], dst_v)
        r0 = s * ROWS_PER_TILE
        pltpu.sync_copy(zeros_hbm.at[pl.ds(r0, ROWS_PER_TILE)],
                        agg_sh.at[pl.ds(r0, ROWS_PER_TILE)])
        plsc.subcore_barrier()

        def chunk(j, carry):
            pltpu.async_copy(x_hbm.at[src_v.at[j]], rows_v, sem).wait()
            pltpu.sync_copy(rows_v, agg_sh.at[dst_v.at[j]], add=True)
            return carry

        lax.fori_loop(0, NCHUNK, chunk, 0)
        plsc.subcore_barrier()
        pltpu.sync_copy(agg_sh.at[pl.ds(r0, ROWS_PER_TILE)],
                        agg_hbm.at[c].at[pl.ds(r0, ROWS_PER_TILE)])

    return k(x, src_r, dst_r, zeros)


def _embed(feats, w, agent_b, task_b):
    br = 1000

    def body(f_ref, w_ref, ba_ref, bt_ref, o_ref):
        i = pl.program_id(0)
        b = jnp.where(i == 0, ba_ref[...], bt_ref[...])
        o_ref[...] = (
            jnp.dot(f_ref[...], w_ref[...], preferred_element_type=jnp.float32) + b
        )

    return pl.pallas_call(
        body,
        grid=(N_NODES // br,),
        in_specs=[
            pl.BlockSpec((br, 16), lambda i: (i, 0)),
            pl.BlockSpec((16, H), lambda i: (0, 0)),
            pl.BlockSpec((1, H), lambda i: (0, 0)),
            pl.BlockSpec((1, H), lambda i: (0, 0)),
        ],
        out_specs=pl.BlockSpec((br, H), lambda i: (i, 0)),
        out_shape=jax.ShapeDtypeStruct((N_NODES, H), jnp.float32),
    )(feats, w, agent_b.reshape(1, H), task_b.reshape(1, H))


def _mlp(x, agg, w1, b1, w2, b2):
    br = 1000

    def body(x_ref, a0_ref, a1_ref, w1_ref, b1_ref, w2_ref, b2_ref, o_ref):
        h = x_ref[...] + a0_ref[0] + a1_ref[0]
        h = jnp.maximum(
            jnp.dot(h, w1_ref[...], preferred_element_type=jnp.float32)
            + b1_ref[...], 0.0)
        h = jnp.dot(h, w2_ref[...], preferred_element_type=jnp.float32) + b2_ref[...]
        o_ref[...] = jnp.maximum(h, 0.0)

    return pl.pallas_call(
        body,
        grid=(N_NODES // br,),
        in_specs=[
            pl.BlockSpec((br, H), lambda i: (i, 0)),
            pl.BlockSpec((1, br, H), lambda i: (0, i, 0)),
            pl.BlockSpec((1, br, H), lambda i: (1, i, 0)),
            pl.BlockSpec((H, H), lambda i: (0, 0)),
            pl.BlockSpec((1, H), lambda i: (0, 0)),
            pl.BlockSpec((H, H), lambda i: (0, 0)),
            pl.BlockSpec((1, H), lambda i: (0, 0)),
        ],
        out_specs=pl.BlockSpec((br, H), lambda i: (i, 0)),
        out_shape=jax.ShapeDtypeStruct((N_NODES, H), jnp.float32),
    )(x, agg, agg, w1, b1.reshape(1, H), w2, b2.reshape(1, H))


def _proj(x, w, b, nrows, row_block_offset):
    br = 1000

    def body(x_ref, w_ref, b_ref, o_ref):
        o_ref[...] = (
            jnp.dot(x_ref[...], w_ref[...], preferred_element_type=jnp.float32)
            + b_ref[...]
        )

    return pl.pallas_call(
        body,
        grid=(nrows // br,),
        in_specs=[
            pl.BlockSpec((br, H), lambda i: (i + row_block_offset, 0)),
            pl.BlockSpec((H, H), lambda i: (0, 0)),
            pl.BlockSpec((1, H), lambda i: (0, 0)),
        ],
        out_specs=pl.BlockSpec((br, H), lambda i: (i, 0)),
        out_shape=jax.ShapeDtypeStruct((nrows, H), jnp.float32),
    )(x, w, b.reshape(1, H))


def kernel(agent_features, task_features, edge_index, agent_in_w, agent_in_b,
           task_in_w, task_in_b, gin_w1, gin_b1, gin_w2, gin_b2,
           agent_out_w, agent_out_b, task_out_w, task_out_b):
    feats = jnp.zeros((N_NODES, 16), jnp.float32)
    feats = feats.at[:NUM_AGENTS, :AF].set(agent_features)
    feats = feats.at[NUM_AGENTS:, AF:AF + TF].set(task_features)
    w_in = jnp.zeros((16, H), jnp.float32)
    w_in = w_in.at[:AF].set(agent_in_w)
    w_in = w_in.at[AF:AF + TF].set(task_in_w)

    pad = E_PAD - N_EDGES
    src_p = jnp.concatenate([edge_index[0], jnp.zeros((pad,), jnp.int32)])
    dst_p = jnp.concatenate(
        [edge_index[1], jnp.full((pad,), TRASH_ROW, jnp.int32)])
    src_r = src_p.reshape(NW, NCHUNK, CHUNK)
    dst_r = dst_p.reshape(NW, NCHUNK, CHUNK)
    zeros = jnp.zeros((AGG_ROWS, H), jnp.float32)

    x = _embed(feats, w_in, agent_in_b, task_in_b)
    for i in range(NUM_LAYERS):
        agg = _sc_scatter(x, src_r, dst_r, zeros)
        x = _mlp(x, agg, gin_w1[i], gin_b1[i], gin_w2[i], gin_b2[i])
    agent_embeddings = _proj(x, agent_out_w, agent_out_b, NUM_AGENTS, 0)
    task_embeddings = _proj(x, task_out_w, task_out_b, NUM_TASKS, 1)
    return (agent_embeddings, task_embeddings)


# speedup vs baseline: 3.6384x; 1.2137x over previous
import functools

import jax
import jax.numpy as jnp
from jax import lax
from jax.experimental import pallas as pl
from jax.experimental.pallas import tpu as pltpu
from jax.experimental.pallas import tpu_sc as plsc

NUM_AGENTS = 1000
NUM_TASKS = 9000
N_NODES = NUM_AGENTS + NUM_TASKS
N_EDGES = 320000
AF = 8
TF = 6
H = 128
NUM_LAYERS = 3

NC = 2
NS = 16
NW = NC * NS
CHUNK = 128
EDGES_PER_TILE = 10240
NCHUNK = EDGES_PER_TILE // CHUNK
NSTAGE = 5
STAGE_CHUNKS = NCHUNK // NSTAGE
REAL_PER_TILE = N_EDGES // NW
PAD_PER_TILE = EDGES_PER_TILE - REAL_PER_TILE
AGG_ROWS = 10240
ROWS_PER_TILE = AGG_ROWS // NS


def _sc_scatter(x, src_r, dst_r, zeros):
    mesh = plsc.VectorSubcoreMesh(
        core_axis_name="c", subcore_axis_name="s", num_cores=NC, num_subcores=NS
    )

    @functools.partial(
        pl.kernel,
        out_type=jax.ShapeDtypeStruct((NC, AGG_ROWS, H), jnp.float32),
        mesh=mesh,
        scratch_types=[
            pltpu.VMEM((STAGE_CHUNKS, CHUNK), jnp.int32),
            pltpu.VMEM((STAGE_CHUNKS, CHUNK), jnp.int32),
            pltpu.VMEM((2, CHUNK, H), jnp.float32),
            pltpu.VMEM_SHARED((AGG_ROWS, H), jnp.float32),
            pltpu.SemaphoreType.DMA,
            pltpu.SemaphoreType.DMA,
        ],
    )
    def k(x_hbm, src_hbm, dst_hbm, zeros_hbm, agg_hbm, src_v, dst_v, rows_v,
          agg_sh, sem0, sem1):
        sems = (sem0, sem1)
        c = lax.axis_index("c")
        s = lax.axis_index("s")
        w = c * NS + s
        r0 = s * ROWS_PER_TILE
        pltpu.sync_copy(zeros_hbm.at[pl.ds(r0, ROWS_PER_TILE)],
                        agg_sh.at[pl.ds(r0, ROWS_PER_TILE)])
        plsc.subcore_barrier()

        def gather_start(j, b):
            pltpu.async_copy(x_hbm.at[src_v.at[j]], rows_v.at[b], sems[b])

        def gather_scatter(j, b):
            pltpu.make_async_copy(x_hbm.at[src_v.at[j]], rows_v.at[b],
                                  sems[b]).wait()
            pltpu.sync_copy(rows_v.at[b], agg_sh.at[dst_v.at[j]], add=True)

        for stage in range(NSTAGE):
            pltpu.sync_copy(
                src_hbm.at[w].at[pl.ds(stage * STAGE_CHUNKS, STAGE_CHUNKS)],
                src_v)
            pltpu.sync_copy(
                dst_hbm.at[w].at[pl.ds(stage * STAGE_CHUNKS, STAGE_CHUNKS)],
                dst_v)
            gather_start(0, 0)
            gather_start(1, 1)

            def step(g, carry):
                for b in range(2):
                    j = 2 * g + b
                    gather_scatter(j, b)
                    gather_start(j + 2, b)
                return carry

            lax.fori_loop(0, STAGE_CHUNKS // 2 - 1, step, 0)
            gather_scatter(STAGE_CHUNKS - 2, 0)
            gather_scatter(STAGE_CHUNKS - 1, 1)

        plsc.subcore_barrier()
        pltpu.sync_copy(agg_sh.at[pl.ds(r0, ROWS_PER_TILE)],
                        agg_hbm.at[c].at[pl.ds(r0, ROWS_PER_TILE)])

    return k(x, src_r, dst_r, zeros)


def _embed(feats, w, agent_b, task_b):
    br = 1000

    def body(f_ref, w_ref, ba_ref, bt_ref, o_ref):
        i = pl.program_id(0)
        b = jnp.where(i == 0, ba_ref[...], bt_ref[...])
        o_ref[...] = (
            jnp.dot(f_ref[...], w_ref[...], preferred_element_type=jnp.float32) + b
        )

    return pl.pallas_call(
        body,
        grid=(N_NODES // br,),
        in_specs=[
            pl.BlockSpec((br, 16), lambda i: (i, 0)),
            pl.BlockSpec((16, H), lambda i: (0, 0)),
            pl.BlockSpec((1, H), lambda i: (0, 0)),
            pl.BlockSpec((1, H), lambda i: (0, 0)),
        ],
        out_specs=pl.BlockSpec((br, H), lambda i: (i, 0)),
        out_shape=jax.ShapeDtypeStruct((N_NODES, H), jnp.float32),
    )(feats, w, agent_b.reshape(1, H), task_b.reshape(1, H))


def _mlp(x, agg, w1, b1, w2, b2):
    br = 1000

    def body(x_ref, a0_ref, a1_ref, w1_ref, b1_ref, w2_ref, b2_ref, o_ref):
        h = x_ref[...] + a0_ref[0] + a1_ref[0]
        h = jnp.maximum(
            jnp.dot(h, w1_ref[...], preferred_element_type=jnp.float32)
            + b1_ref[...], 0.0)
        h = jnp.dot(h, w2_ref[...], preferred_element_type=jnp.float32) + b2_ref[...]
        o_ref[...] = jnp.maximum(h, 0.0)

    return pl.pallas_call(
        body,
        grid=(N_NODES // br,),
        in_specs=[
            pl.BlockSpec((br, H), lambda i: (i, 0)),
            pl.BlockSpec((1, br, H), lambda i: (0, i, 0)),
            pl.BlockSpec((1, br, H), lambda i: (1, i, 0)),
            pl.BlockSpec((H, H), lambda i: (0, 0)),
            pl.BlockSpec((1, H), lambda i: (0, 0)),
            pl.BlockSpec((H, H), lambda i: (0, 0)),
            pl.BlockSpec((1, H), lambda i: (0, 0)),
        ],
        out_specs=pl.BlockSpec((br, H), lambda i: (i, 0)),
        out_shape=jax.ShapeDtypeStruct((N_NODES, H), jnp.float32),
    )(x, agg, agg, w1, b1.reshape(1, H), w2, b2.reshape(1, H))


def _proj(x, w, b, nrows, row_block_offset):
    br = 1000

    def body(x_ref, w_ref, b_ref, o_ref):
        o_ref[...] = (
            jnp.dot(x_ref[...], w_ref[...], preferred_element_type=jnp.float32)
            + b_ref[...]
        )

    return pl.pallas_call(
        body,
        grid=(nrows // br,),
        in_specs=[
            pl.BlockSpec((br, H), lambda i: (i + row_block_offset, 0)),
            pl.BlockSpec((H, H), lambda i: (0, 0)),
            pl.BlockSpec((1, H), lambda i: (0, 0)),
        ],
        out_specs=pl.BlockSpec((br, H), lambda i: (i, 0)),
        out_shape=jax.ShapeDtypeStruct((nrows, H), jnp.float32),
    )(x, w, b.reshape(1, H))


def kernel(agent_features, task_features, edge_index, agent_in_w, agent_in_b,
           task_in_w, task_in_b, gin_w1, gin_b1, gin_w2, gin_b2,
           agent_out_w, agent_out_b, task_out_w, task_out_b):
    feats = jnp.zeros((N_NODES, 16), jnp.float32)
    feats = feats.at[:NUM_AGENTS, :AF].set(agent_features)
    feats = feats.at[NUM_AGENTS:, AF:AF + TF].set(task_features)
    w_in = jnp.zeros((16, H), jnp.float32)
    w_in = w_in.at[:AF].set(agent_in_w)
    w_in = w_in.at[AF:AF + TF].set(task_in_w)

    pad_src = jnp.zeros((NW, PAD_PER_TILE), jnp.int32)
    pad_dst = jnp.broadcast_to(
        N_NODES + jnp.arange(PAD_PER_TILE, dtype=jnp.int32), (NW, PAD_PER_TILE))
    src_r = jnp.concatenate(
        [edge_index[0].reshape(NW, REAL_PER_TILE), pad_src], axis=1
    ).reshape(NW, NCHUNK, CHUNK)
    dst_r = jnp.concatenate(
        [edge_index[1].reshape(NW, REAL_PER_TILE), pad_dst], axis=1
    ).reshape(NW, NCHUNK, CHUNK)
    zeros = jnp.zeros((AGG_ROWS, H), jnp.float32)

    x = _embed(feats, w_in, agent_in_b, task_in_b)
    for i in range(NUM_LAYERS):
        agg = _sc_scatter(x, src_r, dst_r, zeros)
        x = _mlp(x, agg, gin_w1[i], gin_b1[i], gin_w2[i], gin_b2[i])
    agent_embeddings = _proj(x, agent_out_w, agent_out_b, NUM_AGENTS, 0)
    task_embeddings = _proj(x, task_out_w, task_out_b, NUM_TASKS, 1)
    return (agent_embeddings, task_embeddings)


# speedup vs baseline: 3.7191x; 1.0222x over previous
import functools

import jax
import jax.numpy as jnp
from jax import lax
from jax.experimental import pallas as pl
from jax.experimental.pallas import tpu as pltpu
from jax.experimental.pallas import tpu_sc as plsc

NUM_AGENTS = 1000
NUM_TASKS = 9000
N_NODES = NUM_AGENTS + NUM_TASKS
N_EDGES = 320000
AF = 8
TF = 6
H = 128
NUM_LAYERS = 3

NC = 2
NS = 16
NW = NC * NS
CHUNK = 64
NBUF = 4
EDGES_PER_TILE = 10240
NCHUNK = EDGES_PER_TILE // CHUNK
NSTAGE = 5
STAGE_CHUNKS = NCHUNK // NSTAGE
REAL_PER_TILE = N_EDGES // NW
PAD_PER_TILE = EDGES_PER_TILE - REAL_PER_TILE
AGG_ROWS = 10240
ROWS_PER_TILE = AGG_ROWS // NS


def _sc_scatter(x, src_r, dst_r, zeros):
    mesh = plsc.VectorSubcoreMesh(
        core_axis_name="c", subcore_axis_name="s", num_cores=NC, num_subcores=NS
    )

    @functools.partial(
        pl.kernel,
        out_type=jax.ShapeDtypeStruct((NC, AGG_ROWS, H), jnp.float32),
        mesh=mesh,
        scratch_types=[
            pltpu.VMEM((STAGE_CHUNKS, CHUNK), jnp.int32),
            pltpu.VMEM((STAGE_CHUNKS, CHUNK), jnp.int32),
            pltpu.VMEM((NBUF, CHUNK, H), jnp.float32),
            pltpu.VMEM_SHARED((AGG_ROWS, H), jnp.float32),
            pltpu.SemaphoreType.DMA,
            pltpu.SemaphoreType.DMA,
            pltpu.SemaphoreType.DMA,
            pltpu.SemaphoreType.DMA,
        ],
    )
    def k(x_hbm, src_hbm, dst_hbm, zeros_hbm, agg_hbm, src_v, dst_v, rows_v,
          agg_sh, sem0, sem1, sem2, sem3):
        sems = (sem0, sem1, sem2, sem3)
        c = lax.axis_index("c")
        s = lax.axis_index("s")
        w = c * NS + s
        r0 = s * ROWS_PER_TILE
        pltpu.sync_copy(zeros_hbm.at[pl.ds(r0, ROWS_PER_TILE)],
                        agg_sh.at[pl.ds(r0, ROWS_PER_TILE)])
        plsc.subcore_barrier()

        def gather_start(j, b):
            pltpu.async_copy(x_hbm.at[src_v.at[j]], rows_v.at[b], sems[b])

        def gather_scatter(j, b):
            pltpu.make_async_copy(x_hbm.at[src_v.at[j]], rows_v.at[b],
                                  sems[b]).wait()
            pltpu.sync_copy(rows_v.at[b], agg_sh.at[dst_v.at[j]], add=True)

        for stage in range(NSTAGE):
            pltpu.sync_copy(
                src_hbm.at[w].at[pl.ds(stage * STAGE_CHUNKS, STAGE_CHUNKS)],
                src_v)
            pltpu.sync_copy(
                dst_hbm.at[w].at[pl.ds(stage * STAGE_CHUNKS, STAGE_CHUNKS)],
                dst_v)
            for b in range(NBUF):
                gather_start(b, b)

            def step(g, carry):
                for b in range(NBUF):
                    j = NBUF * g + b
                    gather_scatter(j, b)
                    gather_start(j + NBUF, b)
                return carry

            lax.fori_loop(0, STAGE_CHUNKS // NBUF - 1, step, 0)
            for b in range(NBUF):
                gather_scatter(STAGE_CHUNKS - NBUF + b, b)

        plsc.subcore_barrier()
        pltpu.sync_copy(agg_sh.at[pl.ds(r0, ROWS_PER_TILE)],
                        agg_hbm.at[c].at[pl.ds(r0, ROWS_PER_TILE)])

    return k(x, src_r, dst_r, zeros)


def _embed(feats, w, agent_b, task_b):
    br = 1000

    def body(f_ref, w_ref, ba_ref, bt_ref, o_ref):
        i = pl.program_id(0)
        b = jnp.where(i == 0, ba_ref[...], bt_ref[...])
        o_ref[...] = (
            jnp.dot(f_ref[...], w_ref[...], preferred_element_type=jnp.float32) + b
        )

    return pl.pallas_call(
        body,
        grid=(N_NODES // br,),
        in_specs=[
            pl.BlockSpec((br, 16), lambda i: (i, 0)),
            pl.BlockSpec((16, H), lambda i: (0, 0)),
            pl.BlockSpec((1, H), lambda i: (0, 0)),
            pl.BlockSpec((1, H), lambda i: (0, 0)),
        ],
        out_specs=pl.BlockSpec((br, H), lambda i: (i, 0)),
        out_shape=jax.ShapeDtypeStruct((N_NODES, H), jnp.float32),
    )(feats, w, agent_b.reshape(1, H), task_b.reshape(1, H))


def _mlp(x, agg, w1, b1, w2, b2):
    br = 1000

    def body(x_ref, a0_ref, a1_ref, w1_ref, b1_ref, w2_ref, b2_ref, o_ref):
        h = x_ref[...] + a0_ref[0] + a1_ref[0]
        h = jnp.maximum(
            jnp.dot(h, w1_ref[...], preferred_element_type=jnp.float32)
            + b1_ref[...], 0.0)
        h = jnp.dot(h, w2_ref[...], preferred_element_type=jnp.float32) + b2_ref[...]
        o_ref[...] = jnp.maximum(h, 0.0)

    return pl.pallas_call(
        body,
        grid=(N_NODES // br,),
        in_specs=[
            pl.BlockSpec((br, H), lambda i: (i, 0)),
            pl.BlockSpec((1, br, H), lambda i: (0, i, 0)),
            pl.BlockSpec((1, br, H), lambda i: (1, i, 0)),
            pl.BlockSpec((H, H), lambda i: (0, 0)),
            pl.BlockSpec((1, H), lambda i: (0, 0)),
            pl.BlockSpec((H, H), lambda i: (0, 0)),
            pl.BlockSpec((1, H), lambda i: (0, 0)),
        ],
        out_specs=pl.BlockSpec((br, H), lambda i: (i, 0)),
        out_shape=jax.ShapeDtypeStruct((N_NODES, H), jnp.float32),
    )(x, agg, agg, w1, b1.reshape(1, H), w2, b2.reshape(1, H))


def _proj(x, w, b, nrows, row_block_offset):
    br = 1000

    def body(x_ref, w_ref, b_ref, o_ref):
        o_ref[...] = (
            jnp.dot(x_ref[...], w_ref[...], preferred_element_type=jnp.float32)
            + b_ref[...]
        )

    return pl.pallas_call(
        body,
        grid=(nrows // br,),
        in_specs=[
            pl.BlockSpec((br, H), lambda i: (i + row_block_offset, 0)),
            pl.BlockSpec((H, H), lambda i: (0, 0)),
            pl.BlockSpec((1, H), lambda i: (0, 0)),
        ],
        out_specs=pl.BlockSpec((br, H), lambda i: (i, 0)),
        out_shape=jax.ShapeDtypeStruct((nrows, H), jnp.float32),
    )(x, w, b.reshape(1, H))


def kernel(agent_features, task_features, edge_index, agent_in_w, agent_in_b,
           task_in_w, task_in_b, gin_w1, gin_b1, gin_w2, gin_b2,
           agent_out_w, agent_out_b, task_out_w, task_out_b):
    feats = jnp.zeros((N_NODES, 16), jnp.float32)
    feats = feats.at[:NUM_AGENTS, :AF].set(agent_features)
    feats = feats.at[NUM_AGENTS:, AF:AF + TF].set(task_features)
    w_in = jnp.zeros((16, H), jnp.float32)
    w_in = w_in.at[:AF].set(agent_in_w)
    w_in = w_in.at[AF:AF + TF].set(task_in_w)

    pad_src = jnp.zeros((NW, PAD_PER_TILE), jnp.int32)
    pad_dst = jnp.broadcast_to(
        N_NODES + jnp.arange(PAD_PER_TILE, dtype=jnp.int32), (NW, PAD_PER_TILE))
    src_r = jnp.concatenate(
        [edge_index[0].reshape(NW, REAL_PER_TILE), pad_src], axis=1
    ).reshape(NW, NCHUNK, CHUNK)
    dst_r = jnp.concatenate(
        [edge_index[1].reshape(NW, REAL_PER_TILE), pad_dst], axis=1
    ).reshape(NW, NCHUNK, CHUNK)
    zeros = jnp.zeros((AGG_ROWS, H), jnp.float32)

    x = _embed(feats, w_in, agent_in_b, task_in_b)
    for i in range(NUM_LAYERS):
        agg = _sc_scatter(x, src_r, dst_r, zeros)
        x = _mlp(x, agg, gin_w1[i], gin_b1[i], gin_w2[i], gin_b2[i])
    agent_embeddings = _proj(x, agent_out_w, agent_out_b, NUM_AGENTS, 0)
    task_embeddings = _proj(x, task_out_w, task_out_b, NUM_TASKS, 1)
    return (agent_embeddings, task_embeddings)
